# baseline (speedup 1.0000x reference)
import functools

import jax
import jax.numpy as jnp
from jax import lax
from jax.experimental import pallas as pl
from jax.experimental.pallas import tpu as pltpu
from jax.experimental.pallas import tpu_sc as plsc

N_NODES = 10000
D = 256
N_GRAPHS = 512


NC, NS = 2, 16
NW = NC * NS
LANES = 16
CHUNK = 80
NCHUNKS = N_NODES // CHUNK
ITERS = -(-NCHUNKS // NW)

_sc_mesh = plsc.VectorSubcoreMesh(core_axis_name="c", subcore_axis_name="s")


@functools.partial(
    pl.kernel,
    out_type=jax.ShapeDtypeStruct((N_NODES, D), jnp.float32),
    mesh=_sc_mesh,
    scratch_types=[
        pltpu.VMEM((CHUNK,), jnp.int32),
        pltpu.VMEM((CHUNK, D), jnp.float32),
        pltpu.VMEM((CHUNK, D), jnp.float32),
        pltpu.SemaphoreType.DMA,
        pltpu.SemaphoreType.DMA,
    ],
)
def _sc_gather_add(x_hbm, batch_hbm, vx_hbm, h_hbm,
                   idx_v, x_v, rows_v, sem_x, sem_g):
    cid = lax.axis_index("c")
    sid = lax.axis_index("s")
    wid = cid * NS + sid

    @pl.loop(0, ITERS)
    def _(it):
        k = wid + it * NW

        @pl.when(k < NCHUNKS)
        def _():
            base = k * CHUNK
            pltpu.sync_copy(batch_hbm.at[pl.ds(base, CHUNK)], idx_v)
            cp_x = pltpu.async_copy(x_hbm.at[pl.ds(base, CHUNK)], x_v, sem_x)
            cp_g = pltpu.async_copy(vx_hbm.at[idx_v], rows_v, sem_g)
            cp_x.wait()
            cp_g.wait()

            @pl.loop(0, CHUNK, unroll=2)
            def _(i):
                @pl.loop(0, D, step=LANES)
                def _(j):
                    slc = (pl.ds(i, 1), pl.ds(j, LANES))
                    rows_v.at[*slc][...] = rows_v.at[*slc][...] + x_v.at[*slc][...]

            pltpu.sync_copy(rows_v, h_hbm.at[pl.ds(base, CHUNK)])


BLOCK = 400
GRID = N_NODES // BLOCK


def _tc_body(x_ref, batch_ref, vx_ref, W0_ref, W1_ref, bsum_ref, s_ref, t_ref,
             v_ref, pool_acc, cnt_acc):
    i = pl.program_id(0)

    ids = batch_ref[0, 0, :]
    M = (ids[:, None] == lax.broadcasted_iota(jnp.int32, (BLOCK, N_GRAPHS), 1)
         ).astype(jnp.float32)

    part = lax.dot_general(M, x_ref[...], (((0,), (0,)), ((), ())),
                           preferred_element_type=jnp.float32)
    cnt = jnp.sum(M, axis=0).reshape(N_GRAPHS, 1)

    @pl.when(i == 0)
    def _():
        pool_acc[...] = part
        cnt_acc[...] = cnt

    @pl.when(i > 0)
    def _():
        pool_acc[...] += part
        cnt_acc[...] += cnt

    @pl.when(i == GRID - 1)
    def _():
        pooled = pool_acc[...] + cnt_acc[...] * vx_ref[...]
        A = lax.dot_general(vx_ref[...], W0_ref[...], (((1,), (1,)), ((), ())),
                            preferred_element_type=jnp.float32)
        P = lax.dot_general(pooled, W1_ref[...], (((1,), (1,)), ((), ())),
                            preferred_element_type=jnp.float32)
        v = (A + P + bsum_ref[...]) * s_ref[...] + t_ref[...]
        v_ref[...] = jnp.maximum(v, 0.0)


def kernel(x, edge_index, batch, vx, W0_w, W0_b, W1_w, W1_b,
           bn_gamma, bn_beta, bn_mean, bn_var):
    del edge_index
    h = _sc_gather_add(x, batch, vx)

    s = bn_gamma * lax.rsqrt(bn_var + 1e-5)
    t = bn_beta - bn_mean * s
    bsum = (W0_b + W1_b).reshape(1, D)
    batch3 = batch.reshape(GRID, 1, BLOCK)

    v = pl.pallas_call(
        _tc_body,
        grid=(GRID,),
        in_specs=[
            pl.BlockSpec((BLOCK, D), lambda i: (i, 0)),
            pl.BlockSpec((1, 1, BLOCK), lambda i: (i, 0, 0)),
            pl.BlockSpec((N_GRAPHS, D), lambda i: (0, 0)),
            pl.BlockSpec((D, D), lambda i: (0, 0)),
            pl.BlockSpec((D, D), lambda i: (0, 0)),
            pl.BlockSpec((1, D), lambda i: (0, 0)),
            pl.BlockSpec((1, D), lambda i: (0, 0)),
            pl.BlockSpec((1, D), lambda i: (0, 0)),
        ],
        out_specs=pl.BlockSpec((N_GRAPHS, D), lambda i: (0, 0)),
        out_shape=jax.ShapeDtypeStruct((N_GRAPHS, D), jnp.float32),
        scratch_shapes=[
            pltpu.VMEM((N_GRAPHS, D), jnp.float32),
            pltpu.VMEM((N_GRAPHS, 1), jnp.float32),
        ],
    )(x, batch3, vx, W0_w, W1_w, bsum, s.reshape(1, D), t.reshape(1, D))
    return (h, v)


# speedup vs baseline: 1.2796x; 1.2796x over previous
import functools

import jax
import jax.numpy as jnp
from jax import lax
from jax.experimental import pallas as pl
from jax.experimental.pallas import tpu as pltpu
from jax.experimental.pallas import tpu_sc as plsc

N_NODES = 10000
D = 256
N_GRAPHS = 512


NC, NS = 2, 16
NW = NC * NS
LANES = 16
CHUNK = 80
NCHUNKS = N_NODES // CHUNK
ITERS = -(-NCHUNKS // NW)

_sc_mesh = plsc.VectorSubcoreMesh(core_axis_name="c", subcore_axis_name="s")


@functools.partial(
    pl.kernel,
    out_type=jax.ShapeDtypeStruct((N_NODES, D), jnp.float32),
    mesh=_sc_mesh,
    scratch_types=[
        pltpu.VMEM((CHUNK,), jnp.int32),
        pltpu.VMEM((CHUNK, D), jnp.float32),
        pltpu.VMEM((CHUNK, D), jnp.float32),
        pltpu.SemaphoreType.DMA,
        pltpu.SemaphoreType.DMA,
    ],
)
def _sc_gather_add(x_hbm, batch_hbm, vx_hbm, h_hbm,
                   idx_v, x_v, rows_v, sem_x, sem_g):
    cid = lax.axis_index("c")
    sid = lax.axis_index("s")
    wid = cid * NS + sid

    @pl.loop(0, ITERS)
    def _(it):
        k = wid + it * NW

        @pl.when(k < NCHUNKS)
        def _():
            base = k * CHUNK
            pltpu.sync_copy(batch_hbm.at[pl.ds(base, CHUNK)], idx_v)
            cp_x = pltpu.async_copy(x_hbm.at[pl.ds(base, CHUNK)], x_v, sem_x)
            cp_g = pltpu.async_copy(vx_hbm.at[idx_v], rows_v, sem_g)
            cp_x.wait()
            cp_g.wait()

            @pl.loop(0, CHUNK, unroll=2)
            def _(i):
                for j in range(0, D, LANES):
                    slc = (pl.ds(i, 1), pl.ds(j, LANES))
                    rows_v.at[*slc][...] = rows_v.at[*slc][...] + x_v.at[*slc][...]

            pltpu.sync_copy(rows_v, h_hbm.at[pl.ds(base, CHUNK)])


BLOCK = 400
GRID = N_NODES // BLOCK


def _tc_body(x_ref, batch_ref, vx_ref, W0_ref, W1_ref, bsum_ref, s_ref, t_ref,
             v_ref, pool_acc, cnt_acc):
    i = pl.program_id(0)

    ids = batch_ref[0, 0, :]
    M = (ids[:, None] == lax.broadcasted_iota(jnp.int32, (BLOCK, N_GRAPHS), 1)
         ).astype(jnp.float32)

    part = lax.dot_general(M, x_ref[...], (((0,), (0,)), ((), ())),
                           preferred_element_type=jnp.float32)
    cnt = jnp.sum(M, axis=0).reshape(N_GRAPHS, 1)

    @pl.when(i == 0)
    def _():
        pool_acc[...] = part
        cnt_acc[...] = cnt

    @pl.when(i > 0)
    def _():
        pool_acc[...] += part
        cnt_acc[...] += cnt

    @pl.when(i == GRID - 1)
    def _():
        pooled = pool_acc[...] + cnt_acc[...] * vx_ref[...]
        A = lax.dot_general(vx_ref[...], W0_ref[...], (((1,), (1,)), ((), ())),
                            preferred_element_type=jnp.float32)
        P = lax.dot_general(pooled, W1_ref[...], (((1,), (1,)), ((), ())),
                            preferred_element_type=jnp.float32)
        v = (A + P + bsum_ref[...]) * s_ref[...] + t_ref[...]
        v_ref[...] = jnp.maximum(v, 0.0)


def kernel(x, edge_index, batch, vx, W0_w, W0_b, W1_w, W1_b,
           bn_gamma, bn_beta, bn_mean, bn_var):
    del edge_index
    h = _sc_gather_add(x, batch, vx)

    s = bn_gamma * lax.rsqrt(bn_var + 1e-5)
    t = bn_beta - bn_mean * s
    bsum = (W0_b + W1_b).reshape(1, D)
    batch3 = batch.reshape(GRID, 1, BLOCK)

    v = pl.pallas_call(
        _tc_body,
        grid=(GRID,),
        in_specs=[
            pl.BlockSpec((BLOCK, D), lambda i: (i, 0)),
            pl.BlockSpec((1, 1, BLOCK), lambda i: (i, 0, 0)),
            pl.BlockSpec((N_GRAPHS, D), lambda i: (0, 0)),
            pl.BlockSpec((D, D), lambda i: (0, 0)),
            pl.BlockSpec((D, D), lambda i: (0, 0)),
            pl.BlockSpec((1, D), lambda i: (0, 0)),
            pl.BlockSpec((1, D), lambda i: (0, 0)),
            pl.BlockSpec((1, D), lambda i: (0, 0)),
        ],
        out_specs=pl.BlockSpec((N_GRAPHS, D), lambda i: (0, 0)),
        out_shape=jax.ShapeDtypeStruct((N_GRAPHS, D), jnp.float32),
        scratch_shapes=[
            pltpu.VMEM((N_GRAPHS, D), jnp.float32),
            pltpu.VMEM((N_GRAPHS, 1), jnp.float32),
        ],
    )(x, batch3, vx, W0_w, W1_w, bsum, s.reshape(1, D), t.reshape(1, D))
    return (h, v)
